# baseline (speedup 1.0000x reference)
import jax
import jax.numpy as jnp
from jax.experimental import pallas as pl
from jax.experimental.pallas import tpu as pltpu
from jax.experimental.pallas import tpu_sc as plsc

DIM = 64
NC = 2
NS = 16
NW = NC * NS
CHUNK = 128
SCALE = 8.0
NBUF = 4
LOOKAHEAD = 3


def _sc_embed(table, ids3, seq):
    mesh = plsc.VectorSubcoreMesh(
        core_axis_name="c", subcore_axis_name="s", num_cores=NC, num_subcores=NS
    )

    def body(table_hbm, idx_hbm, out_hbm, idx_v, *bufs):
        rows = bufs[:NBUF]
        trans = bufs[NBUF : 2 * NBUF]
        gsem = bufs[2 * NBUF : 3 * NBUF]
        ssem = bufs[3 * NBUF :]
        wid = jax.lax.axis_index("s") * NC + jax.lax.axis_index("c")
        pltpu.sync_copy(idx_hbm.at[:, wid], idx_v)

        t16 = jax.lax.iota(jnp.int32, 16)
        lo3 = jax.lax.shift_right_logical(t16, 3)
        iq1 = jax.lax.bitwise_and(t16, jnp.int32(7))
        iq0 = [2 * q + lo3 for q in range(4)]

        for b in range(LOOKAHEAD):
            pltpu.async_copy(table_hbm.at[idx_v.at[b]], rows[b], gsem[b])

        @pl.loop(0, seq // NBUF)
        def _grp(g):
            for b in range(NBUF):
                j = g * NBUF + b
                jn = j + LOOKAHEAD
                bn = (b + LOOKAHEAD) % NBUF

                @pl.when(jn < seq)
                def _pf():
                    @pl.when(jn >= NBUF)
                    def _w():
                        pltpu.make_async_copy(
                            trans[bn], out_hbm.at[jn - NBUF, :, wid], ssem[bn]
                        ).wait()

                    pltpu.async_copy(table_hbm.at[idx_v.at[jn]], rows[bn], gsem[bn])

                pltpu.make_async_copy(
                    table_hbm.at[idx_v.at[j]], rows[b], gsem[b]
                ).wait()

                @plsc.parallel_loop(0, CHUNK, unroll=8)
                def _r(r):
                    rv = jnp.full((16,), r, jnp.int32)
                    for q in range(4):
                        v = rows[b][r, pl.ds(q * 16, 16)] * SCALE
                        plsc.store_scatter(trans[b], [iq0[q], iq1, rv], v)

                pltpu.async_copy(
                    trans[b].at[:, :, pl.ds(0, CHUNK)], out_hbm.at[j, :, wid], ssem[b]
                )

        for b in range(NBUF):
            pltpu.make_async_copy(
                trans[b].at[:, :, pl.ds(0, CHUNK)],
                out_hbm.at[seq - NBUF + b, :, wid],
                ssem[b],
            ).wait()

    f = pl.kernel(
        body,
        out_type=jax.ShapeDtypeStruct((seq, 8, NW, 8, CHUNK), jnp.float32),
        mesh=mesh,
        compiler_params=pltpu.CompilerParams(
            use_tc_tiling_on_sc=False, needs_layout_passes=False
        ),
        scratch_types=[
            pltpu.VMEM((seq, CHUNK), jnp.int32),
        ]
        + [pltpu.VMEM((CHUNK, DIM), jnp.float32) for _ in range(NBUF)]
        + [pltpu.VMEM((8, 8, CHUNK + 1), jnp.float32) for _ in range(NBUF)]
        + [pltpu.SemaphoreType.DMA for _ in range(2 * NBUF)],
    )
    return f(table, ids3)


TBLK = 512


def _tc_to_row_major(table_t):
    v = table_t.shape[1]
    npair = pl.cdiv(v, 2 * TBLK)

    def body(in_ref, out_ref):
        t = in_ref[...]
        out_ref[...] = jnp.concatenate(
            [t[:, :TBLK].T, t[:, TBLK:].T], axis=1
        )

    return pl.pallas_call(
        body,
        grid=(npair,),
        in_specs=[pl.BlockSpec((DIM, 2 * TBLK), lambda i: (0, i))],
        out_specs=pl.BlockSpec((TBLK, 2 * DIM), lambda i: (i, 0)),
        out_shape=jax.ShapeDtypeStruct((npair * TBLK, 2 * DIM), jnp.float32),
    )(table_t)


def kernel(input_ids, table):
    batch, seq = input_ids.shape
    ids = input_ids.astype(jnp.int32)
    blk = ids >> 9
    ids_p = ((blk >> 1) << 10) + ((ids & (TBLK - 1)) << 1) + (blk & 1)
    ids3 = ids_p.T.reshape(seq, NW, CHUNK)
    t128 = _tc_to_row_major(table.T)
    tlin = t128.reshape(t128.shape[0] * 2, DIM)
    out5 = _sc_embed(tlin, ids3, seq)
    return out5.transpose(2, 4, 0, 1, 3).reshape(batch, seq, DIM)


# speedup vs baseline: 1.3981x; 1.3981x over previous
import jax
import jax.numpy as jnp
from jax.experimental import pallas as pl
from jax.experimental.pallas import tpu as pltpu
from jax.experimental.pallas import tpu_sc as plsc

DIM = 64
NC = 2
NS = 16
NW = NC * NS
CHUNK = 128
SCALE = 8.0
NBUF = 4
LOOKAHEAD = 3


def _sc_embed(table, ids3, seq):
    mesh = plsc.VectorSubcoreMesh(
        core_axis_name="c", subcore_axis_name="s", num_cores=NC, num_subcores=NS
    )

    def body(table_hbm, idx_hbm, out_hbm, idx_v, *bufs):
        rows = bufs[:NBUF]
        trans = bufs[NBUF : 2 * NBUF]
        gsem = bufs[2 * NBUF : 3 * NBUF]
        ssem = bufs[3 * NBUF :]
        wid = jax.lax.axis_index("s") * NC + jax.lax.axis_index("c")
        pltpu.sync_copy(idx_hbm.at[:, wid], idx_v)

        t16 = jax.lax.iota(jnp.int32, 16)
        lo3 = jax.lax.shift_right_logical(t16, 3)
        iq1 = jax.lax.bitwise_and(t16, jnp.int32(7))
        iq0 = [2 * q + lo3 for q in range(4)]

        for b in range(LOOKAHEAD):
            pltpu.async_copy(table_hbm.at[idx_v.at[b]], rows[b], gsem[b])

        @pl.loop(0, seq // NBUF)
        def _grp(g):
            for b in range(NBUF):
                j = g * NBUF + b
                jn = j + LOOKAHEAD
                bn = (b + LOOKAHEAD) % NBUF

                @pl.when(jn < seq)
                def _pf():
                    @pl.when(jn >= NBUF)
                    def _w():
                        pltpu.make_async_copy(
                            trans[bn], out_hbm.at[jn - NBUF, :, wid], ssem[bn]
                        ).wait()

                    pltpu.async_copy(table_hbm.at[idx_v.at[jn]], rows[bn], gsem[bn])

                pltpu.make_async_copy(
                    table_hbm.at[idx_v.at[j]], rows[b], gsem[b]
                ).wait()

                @plsc.parallel_loop(0, CHUNK, unroll=8)
                def _r(r):
                    rv = jnp.full((16,), r, jnp.int32)
                    for q in range(4):
                        v = rows[b][r, pl.ds(q * 16, 16)] * SCALE
                        plsc.store_scatter(trans[b], [iq0[q], iq1, rv], v)

                pltpu.async_copy(
                    trans[b].at[:, :, pl.ds(0, CHUNK)], out_hbm.at[j, :, wid], ssem[b]
                )

        for b in range(NBUF):
            pltpu.make_async_copy(
                trans[b].at[:, :, pl.ds(0, CHUNK)],
                out_hbm.at[seq - NBUF + b, :, wid],
                ssem[b],
            ).wait()

    f = pl.kernel(
        body,
        out_type=jax.ShapeDtypeStruct((seq, 8, NW, 8, CHUNK), jnp.float32),
        mesh=mesh,
        compiler_params=pltpu.CompilerParams(
            use_tc_tiling_on_sc=False, needs_layout_passes=False
        ),
        scratch_types=[
            pltpu.VMEM((seq, CHUNK), jnp.int32),
        ]
        + [pltpu.VMEM((CHUNK, DIM), jnp.float32) for _ in range(NBUF)]
        + [pltpu.VMEM((8, 8, CHUNK + 1), jnp.float32) for _ in range(NBUF)]
        + [pltpu.SemaphoreType.DMA for _ in range(2 * NBUF)],
    )
    return f(table, ids3)


TBLK = 1024


def _tc_to_row_major(table_t):
    v = table_t.shape[1]
    npair = pl.cdiv(v, 2 * TBLK)

    def body(in_ref, out_ref):
        t = in_ref[...]
        out_ref[...] = jnp.concatenate(
            [t[:, :TBLK].T, t[:, TBLK:].T], axis=1
        )

    return pl.pallas_call(
        body,
        grid=(npair,),
        in_specs=[pl.BlockSpec((DIM, 2 * TBLK), lambda i: (0, i))],
        out_specs=pl.BlockSpec((TBLK, 2 * DIM), lambda i: (i, 0)),
        out_shape=jax.ShapeDtypeStruct((npair * TBLK, 2 * DIM), jnp.float32),
    )(table_t)


def kernel(input_ids, table):
    batch, seq = input_ids.shape
    ids = input_ids.astype(jnp.int32)
    tb = TBLK.bit_length() - 1
    blk = ids >> tb
    ids_p = ((blk >> 1) << (tb + 1)) + ((ids & (TBLK - 1)) << 1) + (blk & 1)
    ids3 = ids_p.T.reshape(seq, NW, CHUNK)
    t128 = _tc_to_row_major(table.T)
    tlin = t128.reshape(t128.shape[0] * 2, DIM)
    out5 = _sc_embed(tlin, ids3, seq)
    return out5.transpose(2, 4, 0, 1, 3).reshape(batch, seq, DIM)


# speedup vs baseline: 1.7381x; 1.2432x over previous
import jax
import jax.numpy as jnp
from jax.experimental import pallas as pl
from jax.experimental.pallas import tpu as pltpu
from jax.experimental.pallas import tpu_sc as plsc

DIM = 64
NC = 2
NS = 16
NW = NC * NS
CHUNK = 128
SCALE = 8.0
NBUF = 4
LOOKAHEAD = 3


def _sc_embed(table, ids3, seq):
    mesh = plsc.VectorSubcoreMesh(
        core_axis_name="c", subcore_axis_name="s", num_cores=NC, num_subcores=NS
    )

    def body(table_hbm, idx_hbm, out_hbm, idx_v, *bufs):
        rows = bufs[:NBUF]
        trans = bufs[NBUF : 2 * NBUF]
        gsem = bufs[2 * NBUF : 3 * NBUF]
        ssem = bufs[3 * NBUF :]
        wid = jax.lax.axis_index("s") * NC + jax.lax.axis_index("c")
        pltpu.sync_copy(idx_hbm.at[:, wid], idx_v)

        t16 = jax.lax.iota(jnp.int32, 16)
        lo3 = jax.lax.shift_right_logical(t16, 3)
        iq1 = jax.lax.bitwise_and(t16, jnp.int32(7))
        iq0 = [2 * q + lo3 for q in range(4)]

        for b in range(LOOKAHEAD):
            pltpu.async_copy(table_hbm.at[idx_v.at[b]], rows[b], gsem[b])

        @pl.loop(0, seq // NBUF)
        def _grp(g):
            for b in range(NBUF):
                j = g * NBUF + b
                jn = j + LOOKAHEAD
                bn = (b + LOOKAHEAD) % NBUF

                @pl.when(jn < seq)
                def _pf():
                    @pl.when(jn >= NBUF)
                    def _w():
                        pltpu.make_async_copy(
                            trans[bn], out_hbm.at[jn - NBUF, :, wid], ssem[bn]
                        ).wait()

                    pltpu.async_copy(table_hbm.at[idx_v.at[jn]], rows[bn], gsem[bn])

                pltpu.make_async_copy(
                    table_hbm.at[idx_v.at[j]], rows[b], gsem[b]
                ).wait()

                @plsc.parallel_loop(0, CHUNK, unroll=8)
                def _r(r):
                    rv = jnp.full((16,), r, jnp.int32)
                    for q in range(4):
                        v = rows[b][r, pl.ds(q * 16, 16)] * SCALE
                        plsc.store_scatter(trans[b], [iq0[q], iq1, rv], v)

                pltpu.async_copy(
                    trans[b].at[:, :, pl.ds(0, CHUNK)], out_hbm.at[j, :, wid], ssem[b]
                )

        for b in range(NBUF):
            pltpu.make_async_copy(
                trans[b].at[:, :, pl.ds(0, CHUNK)],
                out_hbm.at[seq - NBUF + b, :, wid],
                ssem[b],
            ).wait()

    f = pl.kernel(
        body,
        out_type=jax.ShapeDtypeStruct((seq, 8, NW, 8, CHUNK), jnp.float32),
        mesh=mesh,
        compiler_params=pltpu.CompilerParams(
            use_tc_tiling_on_sc=False, needs_layout_passes=False
        ),
        scratch_types=[
            pltpu.VMEM((seq, CHUNK), jnp.int32),
        ]
        + [pltpu.VMEM((CHUNK, DIM), jnp.float32) for _ in range(NBUF)]
        + [pltpu.VMEM((8, 8, CHUNK + 1), jnp.float32) for _ in range(NBUF)]
        + [pltpu.SemaphoreType.DMA for _ in range(2 * NBUF)],
    )
    return f(table, ids3)


TBLK = 2048


def _tc_to_row_major(table_t):
    v = table_t.shape[1]
    npair = pl.cdiv(v, 2 * TBLK)

    def body(in_ref, out_ref):
        t = in_ref[...]
        out_ref[...] = jnp.concatenate(
            [t[:, :TBLK].T, t[:, TBLK:].T], axis=1
        )

    return pl.pallas_call(
        body,
        grid=(npair,),
        in_specs=[pl.BlockSpec((DIM, 2 * TBLK), lambda i: (0, i))],
        out_specs=pl.BlockSpec((TBLK, 2 * DIM), lambda i: (i, 0)),
        out_shape=jax.ShapeDtypeStruct((npair * TBLK, 2 * DIM), jnp.float32),
    )(table_t)


def kernel(input_ids, table):
    batch, seq = input_ids.shape
    ids = input_ids.astype(jnp.int32)
    tb = TBLK.bit_length() - 1
    blk = ids >> tb
    ids_p = ((blk >> 1) << (tb + 1)) + ((ids & (TBLK - 1)) << 1) + (blk & 1)
    ids3 = ids_p.T.reshape(seq, NW, CHUNK)
    t128 = _tc_to_row_major(table.T)
    tlin = t128.reshape(t128.shape[0] * 2, DIM)
    out5 = _sc_embed(tlin, ids3, seq)
    return out5.transpose(2, 4, 0, 1, 3).reshape(batch, seq, DIM)


# speedup vs baseline: 2.0100x; 1.1564x over previous
import jax
import jax.numpy as jnp
from jax.experimental import pallas as pl
from jax.experimental.pallas import tpu as pltpu
from jax.experimental.pallas import tpu_sc as plsc

DIM = 64
NC = 2
NS = 16
NW = NC * NS
CHUNK = 128
SCALE = 8.0
NBUF = 4
LOOKAHEAD = 3


def _sc_embed(table, ids3, seq):
    mesh = plsc.VectorSubcoreMesh(
        core_axis_name="c", subcore_axis_name="s", num_cores=NC, num_subcores=NS
    )

    def body(table_hbm, idx_hbm, out_hbm, idx_v, *bufs):
        rows = bufs[:NBUF]
        trans = bufs[NBUF : 2 * NBUF]
        gsem = bufs[2 * NBUF : 3 * NBUF]
        ssem = bufs[3 * NBUF :]
        wid = jax.lax.axis_index("s") * NC + jax.lax.axis_index("c")
        pltpu.sync_copy(idx_hbm.at[:, wid], idx_v)

        t16 = jax.lax.iota(jnp.int32, 16)
        lo3 = jax.lax.shift_right_logical(t16, 3)
        iq1 = jax.lax.bitwise_and(t16, jnp.int32(7))
        iq0 = [2 * q + lo3 for q in range(4)]

        for b in range(LOOKAHEAD):
            pltpu.async_copy(table_hbm.at[idx_v.at[b]], rows[b], gsem[b])

        @pl.loop(0, seq // NBUF)
        def _grp(g):
            for b in range(NBUF):
                j = g * NBUF + b
                jn = j + LOOKAHEAD
                bn = (b + LOOKAHEAD) % NBUF

                @pl.when(jn < seq)
                def _pf():
                    @pl.when(jn >= NBUF)
                    def _w():
                        pltpu.make_async_copy(
                            trans[bn], out_hbm.at[jn - NBUF, :, wid], ssem[bn]
                        ).wait()

                    pltpu.async_copy(table_hbm.at[idx_v.at[jn]], rows[bn], gsem[bn])

                pltpu.make_async_copy(
                    table_hbm.at[idx_v.at[j]], rows[b], gsem[b]
                ).wait()

                @plsc.parallel_loop(0, CHUNK, unroll=8)
                def _r(r):
                    rv = jnp.full((16,), r, jnp.int32)
                    for q in range(4):
                        v = rows[b][r, pl.ds(q * 16, 16)] * SCALE
                        plsc.store_scatter(trans[b], [iq0[q], iq1, rv], v)

                pltpu.async_copy(
                    trans[b].at[:, :, pl.ds(0, CHUNK)], out_hbm.at[j, :, wid], ssem[b]
                )

        for b in range(NBUF):
            pltpu.make_async_copy(
                trans[b].at[:, :, pl.ds(0, CHUNK)],
                out_hbm.at[seq - NBUF + b, :, wid],
                ssem[b],
            ).wait()

    f = pl.kernel(
        body,
        out_type=jax.ShapeDtypeStruct((seq, 8, NW, 8, CHUNK), jnp.float32),
        mesh=mesh,
        compiler_params=pltpu.CompilerParams(
            use_tc_tiling_on_sc=False, needs_layout_passes=False
        ),
        scratch_types=[
            pltpu.VMEM((seq, CHUNK), jnp.int32),
        ]
        + [pltpu.VMEM((CHUNK, DIM), jnp.float32) for _ in range(NBUF)]
        + [pltpu.VMEM((8, 8, CHUNK + 1), jnp.float32) for _ in range(NBUF)]
        + [pltpu.SemaphoreType.DMA for _ in range(2 * NBUF)],
    )
    return f(table, ids3)


TBLK = 4096


def _tc_to_row_major(table_t):
    v = table_t.shape[1]
    npair = pl.cdiv(v, 2 * TBLK)

    def body(in_ref, out_ref):
        t = in_ref[...]
        out_ref[...] = jnp.concatenate(
            [t[:, :TBLK].T, t[:, TBLK:].T], axis=1
        )

    return pl.pallas_call(
        body,
        grid=(npair,),
        in_specs=[pl.BlockSpec((DIM, 2 * TBLK), lambda i: (0, i))],
        out_specs=pl.BlockSpec((TBLK, 2 * DIM), lambda i: (i, 0)),
        out_shape=jax.ShapeDtypeStruct((npair * TBLK, 2 * DIM), jnp.float32),
    )(table_t)


def kernel(input_ids, table):
    batch, seq = input_ids.shape
    ids = input_ids.astype(jnp.int32)
    tb = TBLK.bit_length() - 1
    blk = ids >> tb
    ids_p = ((blk >> 1) << (tb + 1)) + ((ids & (TBLK - 1)) << 1) + (blk & 1)
    ids3 = ids_p.T.reshape(seq, NW, CHUNK)
    t128 = _tc_to_row_major(table.T)
    tlin = t128.reshape(t128.shape[0] * 2, DIM)
    out5 = _sc_embed(tlin, ids3, seq)
    return out5.transpose(2, 4, 0, 1, 3).reshape(batch, seq, DIM)


# speedup vs baseline: 2.1814x; 1.0853x over previous
import jax
import jax.numpy as jnp
from jax.experimental import pallas as pl
from jax.experimental.pallas import tpu as pltpu
from jax.experimental.pallas import tpu_sc as plsc

DIM = 64
NC = 2
NS = 16
NW = NC * NS
CHUNK = 128
SCALE = 8.0
NBUF = 4
LOOKAHEAD = 3


def _sc_embed(table, ids3, seq):
    mesh = plsc.VectorSubcoreMesh(
        core_axis_name="c", subcore_axis_name="s", num_cores=NC, num_subcores=NS
    )

    def body(table_hbm, idx_hbm, out_hbm, idx_v, *bufs):
        rows = bufs[:NBUF]
        trans = bufs[NBUF : 2 * NBUF]
        gsem = bufs[2 * NBUF : 3 * NBUF]
        ssem = bufs[3 * NBUF :]
        wid = jax.lax.axis_index("s") * NC + jax.lax.axis_index("c")
        pltpu.sync_copy(idx_hbm.at[:, wid], idx_v)

        t16 = jax.lax.iota(jnp.int32, 16)
        lo3 = jax.lax.shift_right_logical(t16, 3)
        iq1 = jax.lax.bitwise_and(t16, jnp.int32(7))
        iq0 = [2 * q + lo3 for q in range(4)]

        for b in range(LOOKAHEAD):
            pltpu.async_copy(table_hbm.at[idx_v.at[b]], rows[b], gsem[b])

        @pl.loop(0, seq // NBUF)
        def _grp(g):
            for b in range(NBUF):
                j = g * NBUF + b
                jn = j + LOOKAHEAD
                bn = (b + LOOKAHEAD) % NBUF

                @pl.when(jn < seq)
                def _pf():
                    @pl.when(jn >= NBUF)
                    def _w():
                        pltpu.make_async_copy(
                            trans[bn], out_hbm.at[jn - NBUF, :, wid], ssem[bn]
                        ).wait()

                    pltpu.async_copy(table_hbm.at[idx_v.at[jn]], rows[bn], gsem[bn])

                pltpu.make_async_copy(
                    table_hbm.at[idx_v.at[j]], rows[b], gsem[b]
                ).wait()

                @plsc.parallel_loop(0, CHUNK, unroll=8)
                def _r(r):
                    rv = jnp.full((16,), r, jnp.int32)
                    for q in range(4):
                        v = rows[b][r, pl.ds(q * 16, 16)] * SCALE
                        plsc.store_scatter(trans[b], [iq0[q], iq1, rv], v)

                pltpu.async_copy(
                    trans[b].at[:, :, pl.ds(0, CHUNK)], out_hbm.at[j, :, wid], ssem[b]
                )

        for b in range(NBUF):
            pltpu.make_async_copy(
                trans[b].at[:, :, pl.ds(0, CHUNK)],
                out_hbm.at[seq - NBUF + b, :, wid],
                ssem[b],
            ).wait()

    f = pl.kernel(
        body,
        out_type=jax.ShapeDtypeStruct((seq, 8, NW, 8, CHUNK), jnp.float32),
        mesh=mesh,
        compiler_params=pltpu.CompilerParams(
            use_tc_tiling_on_sc=False, needs_layout_passes=False
        ),
        scratch_types=[
            pltpu.VMEM((seq, CHUNK), jnp.int32),
        ]
        + [pltpu.VMEM((CHUNK, DIM), jnp.float32) for _ in range(NBUF)]
        + [pltpu.VMEM((8, 8, CHUNK + 1), jnp.float32) for _ in range(NBUF)]
        + [pltpu.SemaphoreType.DMA for _ in range(2 * NBUF)],
    )
    return f(table, ids3)


TBLK = 8192


def _tc_to_row_major(table_t):
    v = table_t.shape[1]
    npair = pl.cdiv(v, 2 * TBLK)

    def body(in_ref, out_ref):
        t = in_ref[...]
        out_ref[...] = jnp.concatenate(
            [t[:, :TBLK].T, t[:, TBLK:].T], axis=1
        )

    return pl.pallas_call(
        body,
        grid=(npair,),
        in_specs=[pl.BlockSpec((DIM, 2 * TBLK), lambda i: (0, i))],
        out_specs=pl.BlockSpec((TBLK, 2 * DIM), lambda i: (i, 0)),
        out_shape=jax.ShapeDtypeStruct((npair * TBLK, 2 * DIM), jnp.float32),
    )(table_t)


def kernel(input_ids, table):
    batch, seq = input_ids.shape
    ids = input_ids.astype(jnp.int32)
    tb = TBLK.bit_length() - 1
    blk = ids >> tb
    ids_p = ((blk >> 1) << (tb + 1)) + ((ids & (TBLK - 1)) << 1) + (blk & 1)
    ids3 = ids_p.T.reshape(seq, NW, CHUNK)
    t128 = _tc_to_row_major(table.T)
    tlin = t128.reshape(t128.shape[0] * 2, DIM)
    out5 = _sc_embed(tlin, ids3, seq)
    return out5.transpose(2, 4, 0, 1, 3).reshape(batch, seq, DIM)


# speedup vs baseline: 2.2618x; 1.0368x over previous
import jax
import jax.numpy as jnp
from jax.experimental import pallas as pl
from jax.experimental.pallas import tpu as pltpu
from jax.experimental.pallas import tpu_sc as plsc

DIM = 64
NC = 2
NS = 16
NW = NC * NS
CHUNK = 128
SCALE = 8.0
NBUF = 4
LOOKAHEAD = 3


def _sc_embed(table, ids3, seq):
    mesh = plsc.VectorSubcoreMesh(
        core_axis_name="c", subcore_axis_name="s", num_cores=NC, num_subcores=NS
    )

    def body(table_hbm, idx_hbm, out_hbm, idx_v, *bufs):
        rows = bufs[:NBUF]
        trans = bufs[NBUF : 2 * NBUF]
        gsem = bufs[2 * NBUF : 3 * NBUF]
        ssem = bufs[3 * NBUF :]
        wid = jax.lax.axis_index("s") * NC + jax.lax.axis_index("c")
        pltpu.sync_copy(idx_hbm.at[:, wid], idx_v)

        t16 = jax.lax.iota(jnp.int32, 16)
        lo3 = jax.lax.shift_right_logical(t16, 3)
        iq1 = jax.lax.bitwise_and(t16, jnp.int32(7))
        iq0 = [2 * q + lo3 for q in range(4)]

        for b in range(LOOKAHEAD):
            pltpu.async_copy(table_hbm.at[idx_v.at[b]], rows[b], gsem[b])

        @pl.loop(0, seq // NBUF)
        def _grp(g):
            for b in range(NBUF):
                j = g * NBUF + b
                jn = j + LOOKAHEAD
                bn = (b + LOOKAHEAD) % NBUF

                @pl.when(jn < seq)
                def _pf():
                    @pl.when(jn >= NBUF)
                    def _w():
                        pltpu.make_async_copy(
                            trans[bn], out_hbm.at[jn - NBUF, :, wid], ssem[bn]
                        ).wait()

                    pltpu.async_copy(table_hbm.at[idx_v.at[jn]], rows[bn], gsem[bn])

                pltpu.make_async_copy(
                    table_hbm.at[idx_v.at[j]], rows[b], gsem[b]
                ).wait()

                @plsc.parallel_loop(0, CHUNK, unroll=8)
                def _r(r):
                    rv = jnp.full((16,), r, jnp.int32)
                    for q in range(4):
                        v = rows[b][r, pl.ds(q * 16, 16)] * SCALE
                        plsc.store_scatter(trans[b], [iq0[q], iq1, rv], v)

                pltpu.async_copy(
                    trans[b].at[:, :, pl.ds(0, CHUNK)], out_hbm.at[j, :, wid], ssem[b]
                )

        for b in range(NBUF):
            pltpu.make_async_copy(
                trans[b].at[:, :, pl.ds(0, CHUNK)],
                out_hbm.at[seq - NBUF + b, :, wid],
                ssem[b],
            ).wait()

    f = pl.kernel(
        body,
        out_type=jax.ShapeDtypeStruct((seq, 8, NW, 8, CHUNK), jnp.float32),
        mesh=mesh,
        compiler_params=pltpu.CompilerParams(
            use_tc_tiling_on_sc=False, needs_layout_passes=False
        ),
        scratch_types=[
            pltpu.VMEM((seq, CHUNK), jnp.int32),
        ]
        + [pltpu.VMEM((CHUNK, DIM), jnp.float32) for _ in range(NBUF)]
        + [pltpu.VMEM((8, 8, CHUNK + 1), jnp.float32) for _ in range(NBUF)]
        + [pltpu.SemaphoreType.DMA for _ in range(2 * NBUF)],
    )
    return f(table, ids3)


TBLK = 16384


def _tc_to_row_major(table_t):
    v = table_t.shape[1]
    npair = pl.cdiv(v, 2 * TBLK)

    def body(in_ref, out_ref):
        t = in_ref[...]
        out_ref[...] = jnp.concatenate(
            [t[:, :TBLK].T, t[:, TBLK:].T], axis=1
        )

    return pl.pallas_call(
        body,
        grid=(npair,),
        in_specs=[pl.BlockSpec((DIM, 2 * TBLK), lambda i: (0, i))],
        out_specs=pl.BlockSpec((TBLK, 2 * DIM), lambda i: (i, 0)),
        out_shape=jax.ShapeDtypeStruct((npair * TBLK, 2 * DIM), jnp.float32),
    )(table_t)


def kernel(input_ids, table):
    batch, seq = input_ids.shape
    ids = input_ids.astype(jnp.int32)
    tb = TBLK.bit_length() - 1
    blk = ids >> tb
    ids_p = ((blk >> 1) << (tb + 1)) + ((ids & (TBLK - 1)) << 1) + (blk & 1)
    ids3 = ids_p.T.reshape(seq, NW, CHUNK)
    t128 = _tc_to_row_major(table.T)
    tlin = t128.reshape(t128.shape[0] * 2, DIM)
    out5 = _sc_embed(tlin, ids3, seq)
    return out5.transpose(2, 4, 0, 1, 3).reshape(batch, seq, DIM)
